# baseline (speedup 1.0000x reference)
import functools

import jax
import jax.numpy as jnp
import numpy as np
from jax import lax
from jax.experimental import pallas as pl
from jax.experimental.pallas import tpu as pltpu
from jax.experimental.pallas import tpu_sc as plsc

_NC = 2
_NS = 16
_NW = _NC * _NS

_QCHUNK = 256


def _hilbert_xy(n: int):
    d = np.arange(n * n, dtype=np.int64)
    x = np.zeros(n * n, dtype=np.int64)
    y = np.zeros(n * n, dtype=np.int64)
    t = d.copy()
    s = 1
    while s < n:
        rx = 1 & (t // 2)
        ry = 1 & (t ^ rx)
        swap = ry == 0
        flip = swap & (rx == 1)
        x = np.where(flip, s - 1 - x, x)
        y = np.where(flip, s - 1 - y, y)
        nx = np.where(swap, y, x)
        ny = np.where(swap, x, y)
        x, y = nx, ny
        x = x + s * rx
        y = y + s * ry
        t = t // 4
        s *= 2
    return x, y


@functools.lru_cache(maxsize=None)
def _quadrant_perms(n: int):
    xs, ys = _hilbert_xy(n)
    quads = []
    for q in range(n * n // _QCHUNK):
        cx = xs[q * _QCHUNK:(q + 1) * _QCHUNK]
        cy = ys[q * _QCHUNK:(q + 1) * _QCHUNK]
        x0, y0 = int(cx.min()), int(cy.min())
        assert int(cx.max()) - x0 == 15 and int(cy.max()) - y0 == 15
        quads.append((x0, y0, [(int(a - x0), int(b - y0)) for a, b in zip(cx, cy)]))
    return quads


@functools.lru_cache(maxsize=None)
def _make_permute(b: int, n: int, c: int):
    p = n * n
    quads = _quadrant_perms(n)
    nq = len(quads)
    batches_per_worker = b // _NW
    assert b % _NW == 0
    lanes = 16
    cgroups = c // lanes
    assert c % lanes == 0

    mesh = plsc.VectorSubcoreMesh(core_axis_name="c", subcore_axis_name="s")

    @functools.partial(
        pl.kernel,
        mesh=mesh,
        out_type=jax.ShapeDtypeStruct((b, p, c), jnp.float32),
        scratch_types=[
            pltpu.VMEM((16, 16, c), jnp.float32),
            pltpu.VMEM((_QCHUNK, c), jnp.float32),
        ],
    )
    def permute_kernel(x_hbm, out_hbm, in_v, out_v):
        wid = lax.axis_index("s") * _NC + lax.axis_index("c")
        b0 = wid * batches_per_worker
        for q, (x0, y0, perm) in enumerate(quads):
            def qbody(i, carry, q=q, x0=x0, y0=y0, perm=perm):
                bb = b0 + i
                pltpu.sync_copy(
                    x_hbm.at[bb, pl.ds(x0, 16), pl.ds(y0, 16)], in_v)
                for r, (dx, dy) in enumerate(perm):
                    for k in range(cgroups):
                        out_v[r, pl.ds(k * lanes, lanes)] = (
                            in_v[dx, dy, pl.ds(k * lanes, lanes)])
                pltpu.sync_copy(
                    out_v, out_hbm.at[bb, pl.ds(q * _QCHUNK, _QCHUNK)])
                return carry
            lax.fori_loop(0, batches_per_worker, qbody, 0)

    return permute_kernel


def kernel(inputs):
    b, h, w, c = inputs.shape
    assert h == w
    out = _make_permute(b, h, c)(inputs)
    return out.reshape(b, 1, h * w, c)
